# Initial kernel scaffold; baseline (speedup 1.0000x reference)
import jax
import jax.numpy as jnp
from jax.experimental import pallas as pl


def kernel(hidden_states, w1, w2, router_w):
    raise NotImplementedError("write your pallas kernel here")


# speedup vs baseline: 3.6744x; 3.6744x over previous
import functools
import jax
import jax.numpy as jnp
from jax.experimental import pallas as pl
from jax.experimental.pallas import tpu as pltpu

_H = 768
_I = 1024
_E = 8
_TB = 256


def _moe_body(h_ref, rw_ref, w1_ref, w2_ref, out_ref):
    h = h_ref[...]

    g = jax.lax.dot_general(
        h, rw_ref[...], (((1,), (1,)), ((), ())),
        preferred_element_type=jnp.float32,
    )
    m = jnp.max(g, axis=-1, keepdims=True)
    ex = jnp.exp(g - m)
    p = ex / jnp.sum(ex, axis=-1, keepdims=True)
    eidx = jax.lax.broadcasted_iota(jnp.int32, p.shape, 1)
    m1 = jnp.max(p, axis=-1, keepdims=True)
    i1 = jnp.min(jnp.where(p == m1, eidx, _E), axis=-1, keepdims=True)
    p2 = jnp.where(eidx == i1, -jnp.inf, p)
    m2 = jnp.max(p2, axis=-1, keepdims=True)
    i2 = jnp.min(jnp.where(p2 == m2, eidx, _E), axis=-1, keepdims=True)
    keep = (eidx == i1) | (eidx == i2)
    wmat = jnp.where(keep, p, 0.0)

    hb = h.astype(jnp.bfloat16)
    acc = jnp.zeros((_TB, _H), jnp.float32)
    for e in range(_E):
        x = jax.lax.dot_general(
            hb, w1_ref[e], (((1,), (1,)), ((), ())),
            preferred_element_type=jnp.float32,
        )
        gate = x[:, :_I]
        up = x[:, _I:]
        act = (up * (gate / (1.0 + jnp.exp(-gate)))).astype(jnp.bfloat16)
        y = jax.lax.dot_general(
            act, w2_ref[e], (((1,), (1,)), ((), ())),
            preferred_element_type=jnp.float32,
        )
        acc = acc + y * wmat[:, e:e + 1]
    out_ref[...] = acc


def kernel(hidden_states, w1, w2, router_w):
    orig_shape = hidden_states.shape
    h = hidden_states.reshape(-1, _H)
    T = h.shape[0]
    w1b = w1.astype(jnp.bfloat16)
    w2b = w2.astype(jnp.bfloat16)
    out = pl.pallas_call(
        _moe_body,
        grid=(T // _TB,),
        in_specs=[
            pl.BlockSpec((_TB, _H), lambda i: (i, 0)),
            pl.BlockSpec((_E, _H), lambda i: (0, 0)),
            pl.BlockSpec((_E, 2 * _I, _H), lambda i: (0, 0, 0)),
            pl.BlockSpec((_E, _H, _I), lambda i: (0, 0, 0)),
        ],
        out_specs=pl.BlockSpec((_TB, _H), lambda i: (i, 0)),
        out_shape=jax.ShapeDtypeStruct((T, _H), jnp.float32),
    )(h, router_w, w1b, w2b)
    return out.reshape(orig_shape)
